# baseline (speedup 1.0000x reference)
import functools

import numpy as np
import jax
import jax.numpy as jnp
from jax import lax
from jax.experimental import pallas as pl
from jax.experimental.pallas import tpu as pltpu
from jax.experimental.pallas import tpu_sc as plsc

N = 524288
IN_FEATURES = 3
NUM_LEVELS = 16
FPL = 2
LOG2_T = 19
T = 1 << LOG2_T
MASK = T - 1
BASE_RES = 16
GROWTH = 2.0
PRIME_Y = np.int32(np.uint32(2654435761))
PRIME_Z = np.int32(805459861)

NC = 2
NS = 16
L = 16
NW = NC * NS
P = 512
PTS_W = N // NW
CHUNKS = PTS_W // P
GROUPS = P // L
OUT_F = NUM_LEVELS * FPL
E = 4
D = E * FPL
Q = 8 * P

_LEVELS = []
for _lvl in range(NUM_LEVELS):
    _scale = BASE_RES * (GROWTH ** _lvl) - 1.0
    _res = int(np.ceil(_scale)) + 1
    _LEVELS.append((float(_scale), _res, (_res ** 3) <= T))


def _grid_body(xs_hbm, ys_hbm, zs_hbm, tbl_hbm, out_hbm, xyz_v, xn_v, idx_v,
               col_v, w_v, rows_v, out_v, sem0, sem1):
    wid = lax.axis_index("s") * NC + lax.axis_index("c")
    base_w = wid * PTS_W
    iota = lax.iota(jnp.int32, L)
    sems = (sem0, sem1)

    def phase_a(lvl, sl):
        scale, res, dense = _LEVELS[lvl]
        lvl_base = lvl * T

        def body(g, c2):
            s = pl.ds(g * L, L)
            off = g * L
            pgs = []
            wd = []
            for d in range(3):
                pos = xn_v[d, s] * scale + 0.5
                pg = pos.astype(jnp.int32)
                f = pos - pg.astype(jnp.float32)
                w = f * f * (3.0 - 2.0 * f)
                pgs.append(pg)
                wd.append((1.0 - w, w))
            wxy = [wd[0][a] * wd[1][b] for b in range(2) for a in range(2)]
            for c in range(8):
                w_v[sl, pl.ds(c * P + off, L)] = wxy[c & 3] * wd[2][(c >> 2) & 1]
            if dense:
                cx = (pgs[0], jnp.minimum(pgs[0] + 1, res - 1))
                ay = (pgs[1] * res, jnp.minimum(pgs[1] + 1, res - 1) * res)
                az = (pgs[2] * (res * res) + lvl_base,
                      jnp.minimum(pgs[2] + 1, res - 1) * (res * res) + lvl_base)
                idxs = [cx[c & 1] + ay[(c >> 1) & 1] + az[(c >> 2) & 1]
                        for c in range(8)]
            else:
                hx = (pgs[0], pgs[0] + 1)
                hy0 = pgs[1] * PRIME_Y
                hy = (hy0, hy0 + PRIME_Y)
                hz0 = pgs[2] * PRIME_Z
                hz = (hz0, hz0 + PRIME_Z)
                hxy = [hx[a] ^ hy[b] for b in range(2) for a in range(2)]
                idxs = [((hxy[c & 3] ^ hz[(c >> 2) & 1]) & MASK) + lvl_base
                        for c in range(8)]
            for c in range(8):
                idx_v[sl, pl.ds(c * P + off, L)] = idxs[c] >> 2
                col_v[sl, pl.ds(c * P + off, L)] = (idxs[c] & 3) * 2
            return c2
        lax.fori_loop(0, GROUPS, body, 0)

    def fire(sl):
        return pltpu.async_copy(tbl_hbm.at[idx_v.at[sl]], rows_v.at[sl],
                                sems[sl])

    def phase_b(lvl, sl):
        rows_sl = rows_v.at[sl]

        def body(g, c2):
            off = g * L
            pvec = iota + off
            acc0 = None
            acc1 = None
            for c in range(8):
                qvec = pvec + c * P
                col0 = col_v[sl, pl.ds(c * P + off, L)]
                r0 = plsc.load_gather(rows_sl, [qvec, col0])
                r1 = plsc.load_gather(rows_sl, [qvec, col0 + 1])
                wv = w_v[sl, pl.ds(c * P + off, L)]
                if c == 0:
                    acc0 = r0 * wv
                    acc1 = r1 * wv
                else:
                    acc0 = acc0 + r0 * wv
                    acc1 = acc1 + r1 * wv
            col = jnp.full((L,), 2 * lvl, jnp.int32)
            plsc.store_scatter(out_v, [pvec, col], acc0)
            plsc.store_scatter(out_v, [pvec, col + 1], acc1)
            return c2
        lax.fori_loop(0, GROUPS, body, 0)

    def chunk_body(ch, carry):
        base = base_w + ch * P
        for d, src in enumerate((xs_hbm, ys_hbm, zs_hbm)):
            pltpu.sync_copy(src.at[pl.ds(base, P)], xyz_v.at[d])

        def norm_g(g, c2):
            s = pl.ds(g * L, L)
            for d in range(3):
                xn_v[d, s] = xyz_v[d, s] / 3.0 + 0.5
            return c2
        lax.fori_loop(0, GROUPS, norm_g, 0)

        phase_a(0, 0)
        desc = fire(0)
        for lvl in range(1, NUM_LEVELS):
            sl = lvl & 1
            phase_a(lvl, sl)
            nxt = fire(sl)
            desc.wait()
            phase_b(lvl - 1, 1 - sl)
            desc = nxt
        desc.wait()
        phase_b(NUM_LEVELS - 1, 1)

        pltpu.sync_copy(out_v, out_hbm.at[pl.ds(base, P)])
        return carry

    lax.fori_loop(0, CHUNKS, chunk_body, 0)


_mesh = plsc.VectorSubcoreMesh(core_axis_name="c", subcore_axis_name="s",
                               num_cores=NC, num_subcores=NS)

_grid_kernel = functools.partial(
    pl.kernel,
    out_type=jax.ShapeDtypeStruct((N, OUT_F), jnp.float32),
    mesh=_mesh,
    scratch_types=[
        pltpu.VMEM((3, P), jnp.float32),
        pltpu.VMEM((3, P), jnp.float32),
        pltpu.VMEM((2, Q), jnp.int32),
        pltpu.VMEM((2, Q), jnp.int32),
        pltpu.VMEM((2, Q), jnp.float32),
        pltpu.VMEM((2, Q, D), jnp.float32),
        pltpu.VMEM((P, OUT_F), jnp.float32),
        pltpu.SemaphoreType.DMA,
        pltpu.SemaphoreType.DMA,
    ],
    compiler_params=pltpu.CompilerParams(needs_layout_passes=False,
                                         use_tc_tiling_on_sc=False),
)(_grid_body)


def kernel(points, table):
    xs, ys, zs = (points[:, d] for d in range(3))
    eps = jnp.float32(1e-30)
    tbl = (table.reshape(NUM_LEVELS * T // E, D) + eps) - eps
    return _grid_kernel(xs, ys, zs, tbl)


# speedup vs baseline: 2.4187x; 2.4187x over previous
import functools

import numpy as np
import jax
import jax.numpy as jnp
from jax import lax
from jax.experimental import pallas as pl
from jax.experimental.pallas import tpu as pltpu
from jax.experimental.pallas import tpu_sc as plsc

N = 524288
IN_FEATURES = 3
NUM_LEVELS = 16
FPL = 2
LOG2_T = 19
T = 1 << LOG2_T
MASK = T - 1
BASE_RES = 16
GROWTH = 2.0
PRIME_Y = np.int32(np.uint32(2654435761))
PRIME_Z = np.int32(805459861)

NC = 2
NS = 16
L = 16
NW = NC * NS
P = 256
PTS_W = N // NW
CHUNKS = PTS_W // P
GROUPS = P // L
OUT_F = NUM_LEVELS * FPL
D = 8
Q = 8 * P
LVL_STRIDE = T * FPL

_LEVELS = []
for _lvl in range(NUM_LEVELS):
    _scale = BASE_RES * (GROWTH ** _lvl) - 1.0
    _res = int(np.ceil(_scale)) + 1
    _LEVELS.append((float(_scale), _res, (_res ** 3) <= T))


def _grid_body(xs_hbm, ys_hbm, zs_hbm, tbl_hbm, out_hbm, xyz_v, xn_v, idx_v,
               col_v, w_v, rows_v, out_v, sem0, sem1):
    wid = lax.axis_index("s") * NC + lax.axis_index("c")
    base_w = wid * PTS_W
    iota = lax.iota(jnp.int32, L)
    sems = (sem0, sem1)

    def phase_a(lvl, sl):
        scale, res, dense = _LEVELS[lvl]

        def body(g, c2):
            s = pl.ds(g * L, L)
            off = g * L
            pgs = []
            wd = []
            for d in range(3):
                pos = xn_v[d, s] * scale + 0.5
                pg = pos.astype(jnp.int32)
                f = pos - pg.astype(jnp.float32)
                w = f * f * (3.0 - 2.0 * f)
                pgs.append(pg)
                wd.append((1.0 - w, w))
            wxy = [wd[0][a] * wd[1][b] for b in range(2) for a in range(2)]
            for c in range(8):
                w_v[sl, pl.ds(c * P + off, L)] = wxy[c & 3] * wd[2][(c >> 2) & 1]
            if dense:
                cx = (pgs[0], jnp.minimum(pgs[0] + 1, res - 1))
                ay = (pgs[1] * res, jnp.minimum(pgs[1] + 1, res - 1) * res)
                az = (pgs[2] * (res * res),
                      jnp.minimum(pgs[2] + 1, res - 1) * (res * res))
                idxs = [cx[c & 1] + ay[(c >> 1) & 1] + az[(c >> 2) & 1]
                        for c in range(8)]
            else:
                hx = (pgs[0], pgs[0] + 1)
                hy0 = pgs[1] * PRIME_Y
                hy = (hy0, hy0 + PRIME_Y)
                hz0 = pgs[2] * PRIME_Z
                hz = (hz0, hz0 + PRIME_Z)
                hxy = [hx[a] ^ hy[b] for b in range(2) for a in range(2)]
                idxs = [(hxy[c & 3] ^ hz[(c >> 2) & 1]) & MASK
                        for c in range(8)]
            for c in range(8):
                t = idxs[c]
                a0 = (t + (t & -128)) + lvl * LVL_STRIDE
                row0 = a0 >> 3
                idx_v[sl, pl.ds(c * P + off, L)] = row0
                idx_v[sl, pl.ds(Q + c * P + off, L)] = row0 + 16
                col_v[sl, pl.ds(c * P + off, L)] = a0 & 7
            return c2
        lax.fori_loop(0, GROUPS, body, 0)

    def fire(sl):
        return pltpu.async_copy(tbl_hbm.at[idx_v.at[sl]], rows_v.at[sl],
                                sems[sl])

    def phase_b(lvl, sl):
        rows_sl = rows_v.at[sl]

        def body(g, c2):
            off = g * L
            pvec = iota + off
            acc0 = None
            acc1 = None
            for c in range(8):
                qvec = pvec + c * P
                col0 = col_v[sl, pl.ds(c * P + off, L)]
                r0 = plsc.load_gather(rows_sl, [qvec, col0])
                r1 = plsc.load_gather(rows_sl, [qvec + Q, col0])
                wv = w_v[sl, pl.ds(c * P + off, L)]
                if c == 0:
                    acc0 = r0 * wv
                    acc1 = r1 * wv
                else:
                    acc0 = acc0 + r0 * wv
                    acc1 = acc1 + r1 * wv
            col = jnp.full((L,), 2 * lvl, jnp.int32)
            plsc.store_scatter(out_v, [pvec, col], acc0)
            plsc.store_scatter(out_v, [pvec, col + 1], acc1)
            return c2
        lax.fori_loop(0, GROUPS, body, 0)

    def chunk_body(ch, carry):
        base = base_w + ch * P
        for d, src in enumerate((xs_hbm, ys_hbm, zs_hbm)):
            pltpu.sync_copy(src.at[pl.ds(base, P)], xyz_v.at[d])

        def norm_g(g, c2):
            s = pl.ds(g * L, L)
            for d in range(3):
                xn_v[d, s] = xyz_v[d, s] / 3.0 + 0.5
            return c2
        lax.fori_loop(0, GROUPS, norm_g, 0)

        phase_a(0, 0)
        desc = fire(0)
        for lvl in range(1, NUM_LEVELS):
            sl = lvl & 1
            phase_a(lvl, sl)
            nxt = fire(sl)
            desc.wait()
            phase_b(lvl - 1, 1 - sl)
            desc = nxt
        desc.wait()
        phase_b(NUM_LEVELS - 1, 1)

        pltpu.sync_copy(out_v, out_hbm.at[pl.ds(base, P)])
        return carry

    lax.fori_loop(0, CHUNKS, chunk_body, 0)


_mesh = plsc.VectorSubcoreMesh(core_axis_name="c", subcore_axis_name="s",
                               num_cores=NC, num_subcores=NS)

_grid_kernel = functools.partial(
    pl.kernel,
    out_type=jax.ShapeDtypeStruct((N, OUT_F), jnp.float32),
    mesh=_mesh,
    scratch_types=[
        pltpu.VMEM((3, P), jnp.float32),
        pltpu.VMEM((3, P), jnp.float32),
        pltpu.VMEM((2, 2 * Q), jnp.int32),
        pltpu.VMEM((2, Q), jnp.int32),
        pltpu.VMEM((2, Q), jnp.float32),
        pltpu.VMEM((2, 2 * Q, D), jnp.float32),
        pltpu.VMEM((P, OUT_F), jnp.float32),
        pltpu.SemaphoreType.DMA,
        pltpu.SemaphoreType.DMA,
    ],
    compiler_params=pltpu.CompilerParams(needs_layout_passes=False,
                                         use_tc_tiling_on_sc=False),
)(_grid_body)


def kernel(points, table):
    xs, ys, zs = (points[:, d] for d in range(3))
    tbl = (table.reshape(NUM_LEVELS, T // 128, 128, FPL)
           .transpose(0, 1, 3, 2)
           .reshape(NUM_LEVELS * T * FPL // D, D))
    return _grid_kernel(xs, ys, zs, tbl)


# speedup vs baseline: 2.4361x; 1.0072x over previous
import functools

import numpy as np
import jax
import jax.numpy as jnp
from jax import lax
from jax.experimental import pallas as pl
from jax.experimental.pallas import tpu as pltpu
from jax.experimental.pallas import tpu_sc as plsc

N = 524288
IN_FEATURES = 3
NUM_LEVELS = 16
FPL = 2
LOG2_T = 19
T = 1 << LOG2_T
MASK = T - 1
BASE_RES = 16
GROWTH = 2.0
PRIME_Y = np.int32(np.uint32(2654435761))
PRIME_Z = np.int32(805459861)

NC = 2
NS = 16
L = 16
NW = NC * NS
P = 256
PTS_W = N // NW
CHUNKS = PTS_W // P
GROUPS = P // L
OUT_F = NUM_LEVELS * FPL
D = 8
Q = 8 * P
LVL_STRIDE = T * FPL

_LEVELS = []
for _lvl in range(NUM_LEVELS):
    _scale = BASE_RES * (GROWTH ** _lvl) - 1.0
    _res = int(np.ceil(_scale)) + 1
    _LEVELS.append((float(_scale), _res, (_res ** 3) <= T))


def _grid_body(xs_hbm, ys_hbm, zs_hbm, tbl_hbm, out_hbm, xyz_v, xn_v, idx_v,
               col_v, w_v, rows_v, out_v, sem0, sem1, sem2, sem3):
    wid = lax.axis_index("s") * NC + lax.axis_index("c")
    base_w = wid * PTS_W
    iota = lax.iota(jnp.int32, L)
    sems = ((sem0, sem2), (sem1, sem3))

    def phase_a(lvl, sl):
        scale, res, dense = _LEVELS[lvl]

        def body(g, c2):
            s = pl.ds(g * L, L)
            off = g * L
            pgs = []
            wd = []
            for d in range(3):
                pos = xn_v[d, s] * scale + 0.5
                pg = pos.astype(jnp.int32)
                f = pos - pg.astype(jnp.float32)
                w = f * f * (3.0 - 2.0 * f)
                pgs.append(pg)
                wd.append((1.0 - w, w))
            wxy = [wd[0][a] * wd[1][b] for b in range(2) for a in range(2)]
            for c in range(8):
                w_v[sl, pl.ds(c * P + off, L)] = wxy[c & 3] * wd[2][(c >> 2) & 1]
            if dense:
                cx = (pgs[0], jnp.minimum(pgs[0] + 1, res - 1))
                ay = (pgs[1] * res, jnp.minimum(pgs[1] + 1, res - 1) * res)
                az = (pgs[2] * (res * res),
                      jnp.minimum(pgs[2] + 1, res - 1) * (res * res))
                idxs = [cx[c & 1] + ay[(c >> 1) & 1] + az[(c >> 2) & 1]
                        for c in range(8)]
            else:
                hx = (pgs[0], pgs[0] + 1)
                hy0 = pgs[1] * PRIME_Y
                hy = (hy0, hy0 + PRIME_Y)
                hz0 = pgs[2] * PRIME_Z
                hz = (hz0, hz0 + PRIME_Z)
                hxy = [hx[a] ^ hy[b] for b in range(2) for a in range(2)]
                idxs = [(hxy[c & 3] ^ hz[(c >> 2) & 1]) & MASK
                        for c in range(8)]
            for c in range(8):
                t = idxs[c]
                a0 = (t + (t & -128)) + lvl * LVL_STRIDE
                row0 = a0 >> 3
                idx_v[sl, pl.ds(c * P + off, L)] = row0
                idx_v[sl, pl.ds(Q + c * P + off, L)] = row0 + 16
                col_v[sl, pl.ds(c * P + off, L)] = a0 & 7
            return c2
        lax.fori_loop(0, GROUPS, body, 0)

    def fire(sl):
        return (pltpu.async_copy(tbl_hbm.at[idx_v.at[sl, pl.ds(0, Q)]],
                                 rows_v.at[sl, pl.ds(0, Q)], sems[sl][0]),
                pltpu.async_copy(tbl_hbm.at[idx_v.at[sl, pl.ds(Q, Q)]],
                                 rows_v.at[sl, pl.ds(Q, Q)], sems[sl][1]))

    def phase_b(lvl, sl):
        rows_sl = rows_v.at[sl]

        def body(g, c2):
            off = g * L
            pvec = iota + off
            acc0 = None
            acc1 = None
            for c in range(8):
                qvec = pvec + c * P
                col0 = col_v[sl, pl.ds(c * P + off, L)]
                r0 = plsc.load_gather(rows_sl, [qvec, col0])
                r1 = plsc.load_gather(rows_sl, [qvec + Q, col0])
                wv = w_v[sl, pl.ds(c * P + off, L)]
                if c == 0:
                    acc0 = r0 * wv
                    acc1 = r1 * wv
                else:
                    acc0 = acc0 + r0 * wv
                    acc1 = acc1 + r1 * wv
            col = jnp.full((L,), 2 * lvl, jnp.int32)
            plsc.store_scatter(out_v, [pvec, col], acc0)
            plsc.store_scatter(out_v, [pvec, col + 1], acc1)
            return c2
        lax.fori_loop(0, GROUPS, body, 0)

    def chunk_body(ch, carry):
        base = base_w + ch * P
        for d, src in enumerate((xs_hbm, ys_hbm, zs_hbm)):
            pltpu.sync_copy(src.at[pl.ds(base, P)], xyz_v.at[d])

        def norm_g(g, c2):
            s = pl.ds(g * L, L)
            for d in range(3):
                xn_v[d, s] = xyz_v[d, s] / 3.0 + 0.5
            return c2
        lax.fori_loop(0, GROUPS, norm_g, 0)

        phase_a(0, 0)
        desc = fire(0)
        for lvl in range(1, NUM_LEVELS):
            sl = lvl & 1
            phase_a(lvl, sl)
            nxt = fire(sl)
            desc[0].wait()
            desc[1].wait()
            phase_b(lvl - 1, 1 - sl)
            desc = nxt
        desc[0].wait()
        desc[1].wait()
        phase_b(NUM_LEVELS - 1, 1)

        pltpu.sync_copy(out_v, out_hbm.at[pl.ds(base, P)])
        return carry

    lax.fori_loop(0, CHUNKS, chunk_body, 0)


_mesh = plsc.VectorSubcoreMesh(core_axis_name="c", subcore_axis_name="s",
                               num_cores=NC, num_subcores=NS)

_grid_kernel = functools.partial(
    pl.kernel,
    out_type=jax.ShapeDtypeStruct((N, OUT_F), jnp.float32),
    mesh=_mesh,
    scratch_types=[
        pltpu.VMEM((3, P), jnp.float32),
        pltpu.VMEM((3, P), jnp.float32),
        pltpu.VMEM((2, 2 * Q), jnp.int32),
        pltpu.VMEM((2, Q), jnp.int32),
        pltpu.VMEM((2, Q), jnp.float32),
        pltpu.VMEM((2, 2 * Q, D), jnp.float32),
        pltpu.VMEM((P, OUT_F), jnp.float32),
        pltpu.SemaphoreType.DMA,
        pltpu.SemaphoreType.DMA,
        pltpu.SemaphoreType.DMA,
        pltpu.SemaphoreType.DMA,
    ],
    compiler_params=pltpu.CompilerParams(needs_layout_passes=False,
                                         use_tc_tiling_on_sc=False),
)(_grid_body)


def kernel(points, table):
    xs, ys, zs = (points[:, d] for d in range(3))
    tbl = (table.reshape(NUM_LEVELS, T // 128, 128, FPL)
           .transpose(0, 1, 3, 2)
           .reshape(NUM_LEVELS * T * FPL // D, D))
    return _grid_kernel(xs, ys, zs, tbl)


# speedup vs baseline: 2.8262x; 1.1601x over previous
import functools

import numpy as np
import jax
import jax.numpy as jnp
from jax import lax
from jax.experimental import pallas as pl
from jax.experimental.pallas import tpu as pltpu
from jax.experimental.pallas import tpu_sc as plsc

N = 524288
IN_FEATURES = 3
NUM_LEVELS = 16
FPL = 2
LOG2_T = 19
T = 1 << LOG2_T
MASK = T - 1
BASE_RES = 16
GROWTH = 2.0
PRIME_Y = np.int32(np.uint32(2654435761))
PRIME_Z = np.int32(805459861)

NC = 2
NS = 16
L = 16
NW = NC * NS
P = 256
PTS_W = N // NW
CHUNKS = PTS_W // P
GROUPS = P // L
OUT_F = NUM_LEVELS * FPL
D = 8
Q = 8 * P
LVL_STRIDE = T * FPL

_LEVELS = []
for _lvl in range(NUM_LEVELS):
    _scale = BASE_RES * (GROWTH ** _lvl) - 1.0
    _res = int(np.ceil(_scale)) + 1
    _LEVELS.append((float(_scale), _res, (_res ** 3) <= T))


def _grid_body(xs_hbm, ys_hbm, zs_hbm, tbl_hbm, out_hbm, xyz_v, xn_v, idx_v,
               col_v, w_v, rows_v, out_v, tbl0_v, sem0, sem1, sem2, sem3):
    wid = lax.axis_index("s") * NC + lax.axis_index("c")
    base_w = wid * PTS_W
    iota = lax.iota(jnp.int32, L)
    sems = ((sem0, sem2), (sem1, sem3))

    def phase_a(lvl, sl):
        scale, res, dense = _LEVELS[lvl]

        def body(g, c2):
            s = pl.ds(g * L, L)
            off = g * L
            pgs = []
            wd = []
            for d in range(3):
                pos = xn_v[d, s] * scale + 0.5
                pg = pos.astype(jnp.int32)
                f = pos - pg.astype(jnp.float32)
                w = f * f * (3.0 - 2.0 * f)
                pgs.append(pg)
                wd.append((1.0 - w, w))
            wxy = [wd[0][a] * wd[1][b] for b in range(2) for a in range(2)]
            for c in range(8):
                w_v[sl, pl.ds(c * P + off, L)] = wxy[c & 3] * wd[2][(c >> 2) & 1]
            if dense:
                cx = (pgs[0], jnp.minimum(pgs[0] + 1, res - 1))
                ay = (pgs[1] * res, jnp.minimum(pgs[1] + 1, res - 1) * res)
                az = (pgs[2] * (res * res),
                      jnp.minimum(pgs[2] + 1, res - 1) * (res * res))
                idxs = [cx[c & 1] + ay[(c >> 1) & 1] + az[(c >> 2) & 1]
                        for c in range(8)]
            else:
                hx = (pgs[0], pgs[0] + 1)
                hy0 = pgs[1] * PRIME_Y
                hy = (hy0, hy0 + PRIME_Y)
                hz0 = pgs[2] * PRIME_Z
                hz = (hz0, hz0 + PRIME_Z)
                hxy = [hx[a] ^ hy[b] for b in range(2) for a in range(2)]
                idxs = [(hxy[c & 3] ^ hz[(c >> 2) & 1]) & MASK
                        for c in range(8)]
            for c in range(8):
                t = idxs[c]
                a0 = (t + (t & -128)) + lvl * LVL_STRIDE
                row0 = a0 >> 3
                idx_v[sl, pl.ds(c * P + off, L)] = row0
                idx_v[sl, pl.ds(Q + c * P + off, L)] = row0 + 16
                col_v[sl, pl.ds(c * P + off, L)] = a0 & 7
            return c2
        lax.fori_loop(0, GROUPS, body, 0)

    def fire(sl):
        return (pltpu.async_copy(tbl_hbm.at[idx_v.at[sl, pl.ds(0, Q)]],
                                 rows_v.at[sl, pl.ds(0, Q)], sems[sl][0]),
                pltpu.async_copy(tbl_hbm.at[idx_v.at[sl, pl.ds(Q, Q)]],
                                 rows_v.at[sl, pl.ds(Q, Q)], sems[sl][1]))

    def phase_b0(lvl, sl):
        def body(g, c2):
            off = g * L
            pvec = iota + off
            acc0 = None
            acc1 = None
            for c in range(8):
                rvec = idx_v[sl, pl.ds(c * P + off, L)]
                col0 = col_v[sl, pl.ds(c * P + off, L)]
                r0 = plsc.load_gather(tbl0_v, [rvec, col0])
                r1 = plsc.load_gather(tbl0_v, [rvec + 16, col0])
                wv = w_v[sl, pl.ds(c * P + off, L)]
                if c == 0:
                    acc0 = r0 * wv
                    acc1 = r1 * wv
                else:
                    acc0 = acc0 + r0 * wv
                    acc1 = acc1 + r1 * wv
            col = jnp.full((L,), 2 * lvl, jnp.int32)
            plsc.store_scatter(out_v, [pvec, col], acc0)
            plsc.store_scatter(out_v, [pvec, col + 1], acc1)
            return c2
        lax.fori_loop(0, GROUPS, body, 0)

    def phase_b(lvl, sl):
        rows_sl = rows_v.at[sl]

        def body(g, c2):
            off = g * L
            pvec = iota + off
            acc0 = None
            acc1 = None
            for c in range(8):
                qvec = pvec + c * P
                col0 = col_v[sl, pl.ds(c * P + off, L)]
                r0 = plsc.load_gather(rows_sl, [qvec, col0])
                r1 = plsc.load_gather(rows_sl, [qvec + Q, col0])
                wv = w_v[sl, pl.ds(c * P + off, L)]
                if c == 0:
                    acc0 = r0 * wv
                    acc1 = r1 * wv
                else:
                    acc0 = acc0 + r0 * wv
                    acc1 = acc1 + r1 * wv
            col = jnp.full((L,), 2 * lvl, jnp.int32)
            plsc.store_scatter(out_v, [pvec, col], acc0)
            plsc.store_scatter(out_v, [pvec, col + 1], acc1)
            return c2
        lax.fori_loop(0, GROUPS, body, 0)

    pltpu.sync_copy(tbl_hbm.at[pl.ds(0, 1024)], tbl0_v)

    def chunk_body(ch, carry):
        base = base_w + ch * P
        for d, src in enumerate((xs_hbm, ys_hbm, zs_hbm)):
            pltpu.sync_copy(src.at[pl.ds(base, P)], xyz_v.at[d])

        def norm_g(g, c2):
            s = pl.ds(g * L, L)
            for d in range(3):
                xn_v[d, s] = xyz_v[d, s] / 3.0 + 0.5
            return c2
        lax.fori_loop(0, GROUPS, norm_g, 0)

        phase_a(0, 0)
        phase_b0(0, 0)
        phase_a(1, 1)
        desc = fire(1)
        for lvl in range(2, NUM_LEVELS):
            sl = lvl & 1
            phase_a(lvl, sl)
            nxt = fire(sl)
            desc[0].wait()
            desc[1].wait()
            phase_b(lvl - 1, 1 - sl)
            desc = nxt
        desc[0].wait()
        desc[1].wait()
        phase_b(NUM_LEVELS - 1, 1)

        pltpu.sync_copy(out_v, out_hbm.at[pl.ds(base, P)])
        return carry

    lax.fori_loop(0, CHUNKS, chunk_body, 0)


_mesh = plsc.VectorSubcoreMesh(core_axis_name="c", subcore_axis_name="s",
                               num_cores=NC, num_subcores=NS)

_grid_kernel = functools.partial(
    pl.kernel,
    out_type=jax.ShapeDtypeStruct((N, OUT_F), jnp.float32),
    mesh=_mesh,
    scratch_types=[
        pltpu.VMEM((3, P), jnp.float32),
        pltpu.VMEM((3, P), jnp.float32),
        pltpu.VMEM((2, 2 * Q), jnp.int32),
        pltpu.VMEM((2, Q), jnp.int32),
        pltpu.VMEM((2, Q), jnp.float32),
        pltpu.VMEM((2, 2 * Q, D), jnp.float32),
        pltpu.VMEM((P, OUT_F), jnp.float32),
        pltpu.VMEM((1024, D), jnp.float32),
        pltpu.SemaphoreType.DMA,
        pltpu.SemaphoreType.DMA,
        pltpu.SemaphoreType.DMA,
        pltpu.SemaphoreType.DMA,
    ],
    compiler_params=pltpu.CompilerParams(needs_layout_passes=False,
                                         use_tc_tiling_on_sc=False),
)(_grid_body)


def kernel(points, table):
    xs, ys, zs = (points[:, d] for d in range(3))
    tbl = (table.reshape(NUM_LEVELS, T // 128, 128, FPL)
           .transpose(0, 1, 3, 2)
           .reshape(NUM_LEVELS * T * FPL // D, D))
    return _grid_kernel(xs, ys, zs, tbl)


# speedup vs baseline: 3.0251x; 1.0704x over previous
import functools

import numpy as np
import jax
import jax.numpy as jnp
from jax import lax
from jax.experimental import pallas as pl
from jax.experimental.pallas import tpu as pltpu
from jax.experimental.pallas import tpu_sc as plsc

N = 524288
IN_FEATURES = 3
NUM_LEVELS = 16
FPL = 2
LOG2_T = 19
T = 1 << LOG2_T
MASK = T - 1
BASE_RES = 16
GROWTH = 2.0
PRIME_Y = np.int32(np.uint32(2654435761))
PRIME_Z = np.int32(805459861)

NC = 2
NS = 16
L = 16
NW = NC * NS
P = 128
PTS_W = N // NW
CHUNKS = PTS_W // P
GROUPS = P // L
OUT_F = NUM_LEVELS * FPL
D = 8
Q = 8 * P
LVL_STRIDE = T * FPL

_LEVELS = []
for _lvl in range(NUM_LEVELS):
    _scale = BASE_RES * (GROWTH ** _lvl) - 1.0
    _res = int(np.ceil(_scale)) + 1
    _LEVELS.append((float(_scale), _res, (_res ** 3) <= T))


def _grid_body(xs_hbm, ys_hbm, zs_hbm, tbl_hbm, out_hbm, xyz_v, xn_v, idx_v,
               col_v, w_v, rows_v, out_v, tbl0_v, tbl1_v, sem0, sem1, sem2,
               sem3):
    wid = lax.axis_index("s") * NC + lax.axis_index("c")
    base_w = wid * PTS_W
    iota = lax.iota(jnp.int32, L)
    sems = ((sem0, sem2), (sem1, sem3))

    def phase_a(lvl, sl):
        scale, res, dense = _LEVELS[lvl]

        def body(g, c2):
            s = pl.ds(g * L, L)
            off = g * L
            pgs = []
            wd = []
            for d in range(3):
                pos = xn_v[d, s] * scale + 0.5
                pg = pos.astype(jnp.int32)
                f = pos - pg.astype(jnp.float32)
                w = f * f * (3.0 - 2.0 * f)
                pgs.append(pg)
                wd.append((1.0 - w, w))
            wxy = [wd[0][a] * wd[1][b] for b in range(2) for a in range(2)]
            for c in range(8):
                w_v[sl, pl.ds(c * P + off, L)] = wxy[c & 3] * wd[2][(c >> 2) & 1]
            if dense:
                cx = (pgs[0], jnp.minimum(pgs[0] + 1, res - 1))
                ay = (pgs[1] * res, jnp.minimum(pgs[1] + 1, res - 1) * res)
                az = (pgs[2] * (res * res),
                      jnp.minimum(pgs[2] + 1, res - 1) * (res * res))
                idxs = [cx[c & 1] + ay[(c >> 1) & 1] + az[(c >> 2) & 1]
                        for c in range(8)]
            else:
                hx = (pgs[0], pgs[0] + 1)
                hy0 = pgs[1] * PRIME_Y
                hy = (hy0, hy0 + PRIME_Y)
                hz0 = pgs[2] * PRIME_Z
                hz = (hz0, hz0 + PRIME_Z)
                hxy = [hx[a] ^ hy[b] for b in range(2) for a in range(2)]
                idxs = [(hxy[c & 3] ^ hz[(c >> 2) & 1]) & MASK
                        for c in range(8)]
            for c in range(8):
                t = idxs[c]
                a0 = (t + (t & -128)) + (0 if lvl <= 1 else lvl * LVL_STRIDE)
                row0 = a0 >> 3
                idx_v[sl, pl.ds(c * P + off, L)] = row0
                idx_v[sl, pl.ds(Q + c * P + off, L)] = row0 + 16
                col_v[sl, pl.ds(c * P + off, L)] = a0 & 7
            return c2
        lax.fori_loop(0, GROUPS, body, 0)

    def fire(sl):
        return (pltpu.async_copy(tbl_hbm.at[idx_v.at[sl, pl.ds(0, Q)]],
                                 rows_v.at[sl, pl.ds(0, Q)], sems[sl][0]),
                pltpu.async_copy(tbl_hbm.at[idx_v.at[sl, pl.ds(Q, Q)]],
                                 rows_v.at[sl, pl.ds(Q, Q)], sems[sl][1]))

    def phase_b0(lvl, sl, tblv):
        def body(g, c2):
            off = g * L
            pvec = iota + off
            acc0 = None
            acc1 = None
            for c in range(8):
                rvec = idx_v[sl, pl.ds(c * P + off, L)]
                col0 = col_v[sl, pl.ds(c * P + off, L)]
                r0 = plsc.load_gather(tblv, [rvec, col0])
                r1 = plsc.load_gather(tblv, [rvec + 16, col0])
                wv = w_v[sl, pl.ds(c * P + off, L)]
                if c == 0:
                    acc0 = r0 * wv
                    acc1 = r1 * wv
                else:
                    acc0 = acc0 + r0 * wv
                    acc1 = acc1 + r1 * wv
            col = jnp.full((L,), 2 * lvl, jnp.int32)
            plsc.store_scatter(out_v, [pvec, col], acc0)
            plsc.store_scatter(out_v, [pvec, col + 1], acc1)
            return c2
        lax.fori_loop(0, GROUPS, body, 0)

    def phase_b(lvl, sl):
        rows_sl = rows_v.at[sl]

        def body(g, c2):
            off = g * L
            pvec = iota + off
            acc0 = None
            acc1 = None
            for c in range(8):
                qvec = pvec + c * P
                col0 = col_v[sl, pl.ds(c * P + off, L)]
                r0 = plsc.load_gather(rows_sl, [qvec, col0])
                r1 = plsc.load_gather(rows_sl, [qvec + Q, col0])
                wv = w_v[sl, pl.ds(c * P + off, L)]
                if c == 0:
                    acc0 = r0 * wv
                    acc1 = r1 * wv
                else:
                    acc0 = acc0 + r0 * wv
                    acc1 = acc1 + r1 * wv
            col = jnp.full((L,), 2 * lvl, jnp.int32)
            plsc.store_scatter(out_v, [pvec, col], acc0)
            plsc.store_scatter(out_v, [pvec, col + 1], acc1)
            return c2
        lax.fori_loop(0, GROUPS, body, 0)

    pltpu.sync_copy(tbl_hbm.at[pl.ds(0, 1024)], tbl0_v)
    pltpu.sync_copy(tbl_hbm.at[pl.ds(LVL_STRIDE // D, 8192)], tbl1_v)

    def chunk_body(ch, carry):
        base = base_w + ch * P
        for d, src in enumerate((xs_hbm, ys_hbm, zs_hbm)):
            pltpu.sync_copy(src.at[pl.ds(base, P)], xyz_v.at[d])

        def norm_g(g, c2):
            s = pl.ds(g * L, L)
            for d in range(3):
                xn_v[d, s] = xyz_v[d, s] / 3.0 + 0.5
            return c2
        lax.fori_loop(0, GROUPS, norm_g, 0)

        phase_a(0, 0)
        phase_b0(0, 0, tbl0_v)
        phase_a(1, 1)
        phase_b0(1, 1, tbl1_v)
        phase_a(2, 0)
        desc = fire(0)
        for lvl in range(3, NUM_LEVELS):
            sl = lvl & 1
            phase_a(lvl, sl)
            nxt = fire(sl)
            desc[0].wait()
            desc[1].wait()
            phase_b(lvl - 1, 1 - sl)
            desc = nxt
        desc[0].wait()
        desc[1].wait()
        phase_b(NUM_LEVELS - 1, 1)

        pltpu.sync_copy(out_v, out_hbm.at[pl.ds(base, P)])
        return carry

    lax.fori_loop(0, CHUNKS, chunk_body, 0)


_mesh = plsc.VectorSubcoreMesh(core_axis_name="c", subcore_axis_name="s",
                               num_cores=NC, num_subcores=NS)

_grid_kernel = functools.partial(
    pl.kernel,
    out_type=jax.ShapeDtypeStruct((N, OUT_F), jnp.float32),
    mesh=_mesh,
    scratch_types=[
        pltpu.VMEM((3, P), jnp.float32),
        pltpu.VMEM((3, P), jnp.float32),
        pltpu.VMEM((2, 2 * Q), jnp.int32),
        pltpu.VMEM((2, Q), jnp.int32),
        pltpu.VMEM((2, Q), jnp.float32),
        pltpu.VMEM((2, 2 * Q, D), jnp.float32),
        pltpu.VMEM((P, OUT_F), jnp.float32),
        pltpu.VMEM((1024, D), jnp.float32),
        pltpu.VMEM((8192, D), jnp.float32),
        pltpu.SemaphoreType.DMA,
        pltpu.SemaphoreType.DMA,
        pltpu.SemaphoreType.DMA,
        pltpu.SemaphoreType.DMA,
    ],
    compiler_params=pltpu.CompilerParams(needs_layout_passes=False,
                                         use_tc_tiling_on_sc=False),
)(_grid_body)


def kernel(points, table):
    xs, ys, zs = (points[:, d] for d in range(3))
    tbl = (table.reshape(NUM_LEVELS, T // 128, 128, FPL)
           .transpose(0, 1, 3, 2)
           .reshape(NUM_LEVELS * T * FPL // D, D))
    return _grid_kernel(xs, ys, zs, tbl)
